# baseline (speedup 1.0000x reference)
import functools

import jax
import jax.numpy as jnp
from jax import lax
from jax.experimental import pallas as pl
from jax.experimental.pallas import tpu as pltpu
from jax.experimental.pallas import tpu_sc as plsc

_F32 = jnp.float32
_BATCH = 128
_NTILES = 32


def _pad_edges(src, dst, n_batches, dummy_row, n_dummy):
    e = src.shape[0]
    pad = n_batches * _BATCH - e
    src_p = jnp.concatenate([src, jnp.zeros((pad,), jnp.int32)])
    dst_fill = dummy_row + (jnp.arange(pad, dtype=jnp.int32) % n_dummy)
    dst_p = jnp.concatenate([dst, dst_fill])
    return src_p.reshape(n_batches, _BATCH), dst_p.reshape(n_batches, _BATCH)


def _sc_segment_sum(table, src2d, dst2d, n_rows_acc, feat, nbuf):
    n_batches = src2d.shape[0]
    nb_per_tile = n_batches // _NTILES
    nring = 2 * nbuf
    assert nb_per_tile % nring == 0
    chunk = n_rows_acc // 16
    zeros = jnp.zeros((chunk, feat), _F32)
    mesh = plsc.VectorSubcoreMesh(core_axis_name="c", subcore_axis_name="s")

    @functools.partial(
        pl.kernel,
        mesh=mesh,
        out_type=jax.ShapeDtypeStruct((2, n_rows_acc, feat), _F32),
        scratch_types=[
            pltpu.VMEM((nring, _BATCH), jnp.int32),
            pltpu.VMEM((nring, _BATCH), jnp.int32),
            pltpu.VMEM((nbuf, _BATCH, feat), _F32),
            pltpu.VMEM_SHARED((n_rows_acc, feat), _F32),
            [pltpu.SemaphoreType.DMA] * nring,
            [pltpu.SemaphoreType.DMA] * nbuf,
            [pltpu.SemaphoreType.DMA] * nbuf,
        ],
    )
    def k(table_h, src_h, dst_h, zeros_h, out_h, src_v, dst_v, rows_v, acc,
          isems, gsems, ssems):
        c = lax.axis_index("c")
        s = lax.axis_index("s")
        wid = (1 - c) * 16 + s
        base = wid * nb_per_tile

        def idx_start(j, slot):
            off = (base + j) * _BATCH
            pltpu.async_copy(
                src_h.at[pl.ds(off, _BATCH)], src_v.at[slot], isems[slot])
            pltpu.async_copy(
                dst_h.at[pl.ds(off, _BATCH)], dst_v.at[slot], isems[slot])

        def idx_wait(j, slot):
            off = (base + j) * _BATCH
            pltpu.make_async_copy(
                src_h.at[pl.ds(off, _BATCH)], src_v.at[slot],
                isems[slot]).wait()
            pltpu.make_async_copy(
                dst_h.at[pl.ds(off, _BATCH)], dst_v.at[slot],
                isems[slot]).wait()

        pltpu.sync_copy(zeros_h, acc.at[pl.ds(s * chunk, chunk)])
        for m in range(nring):
            idx_start(m, m)
        for b in range(nbuf):
            idx_wait(b, b)
            pltpu.async_copy(table_h.at[src_v.at[b]], rows_v.at[b], gsems[b])
        plsc.subcore_barrier()

        def body(g, carry):
            j0 = g * nring
            for p in range(2):
                for b in range(nbuf):
                    j = j0 + p * nbuf + b
                    slot = p * nbuf + b
                    pltpu.make_async_copy(
                        table_h.at[src_v.at[slot]], rows_v.at[b],
                        gsems[b]).wait()
                    pltpu.async_copy(
                        rows_v.at[b], acc.at[dst_v.at[slot]], ssems[b],
                        add=True)
                for b in range(nbuf):
                    j = j0 + p * nbuf + b
                    slot = p * nbuf + b
                    nslot = (slot + nbuf) % nring
                    pltpu.make_async_copy(
                        rows_v.at[b], acc.at[dst_v.at[slot]], ssems[b]).wait()
                    nj = j + nbuf

                    @pl.when(nj < nb_per_tile)
                    def _():
                        idx_wait(nj, nslot)
                        pltpu.async_copy(
                            table_h.at[src_v.at[nslot]], rows_v.at[b],
                            gsems[b])

                    nnj = j + nring

                    @pl.when(nnj < nb_per_tile)
                    def _():
                        idx_start(nnj, slot)
            return carry

        lax.fori_loop(0, nb_per_tile // nring, body, 0)
        plsc.subcore_barrier()
        pltpu.sync_copy(acc.at[pl.ds(s * chunk, chunk)],
                        out_h.at[c].at[pl.ds(s * chunk, chunk)])

    return k(table, src2d.reshape(-1), dst2d.reshape(-1), zeros)


def _sc_degree(dst_flat, n_rows_acc):
    n_batches = dst_flat.shape[0] // _BATCH
    nb_per_tile = n_batches // _NTILES
    e_per_tile = nb_per_tile * _BATCH
    mesh = plsc.VectorSubcoreMesh(core_axis_name="c", subcore_axis_name="s")

    @functools.partial(
        pl.kernel,
        mesh=mesh,
        out_type=jax.ShapeDtypeStruct((_NTILES, n_rows_acc), _F32),
        scratch_types=[
            pltpu.VMEM((e_per_tile,), jnp.int32),
            pltpu.VMEM((n_rows_acc,), _F32),
        ],
        compiler_params=pltpu.CompilerParams(needs_layout_passes=False),
    )
    def k(dst_h, out_h, dst_v, acc_v):
        c = lax.axis_index("c")
        s = lax.axis_index("s")
        wid = c * 16 + s
        pltpu.sync_copy(dst_h.at[pl.ds(wid * e_per_tile, e_per_tile)], dst_v)
        zero16 = jnp.zeros((16,), _F32)

        def zbody(i, carry):
            acc_v[pl.ds(i * 16, 16)] = zero16
            return carry

        lax.fori_loop(0, n_rows_acc // 16, zbody, 0)
        one16 = jnp.ones((16,), _F32)

        def body(j, carry):
            idx = dst_v[pl.ds(j * 16, 16)]
            plsc.addupdate_scatter(acc_v, [idx], one16)
            return carry

        lax.fori_loop(0, e_per_tile // 16, body, 0)
        pltpu.sync_copy(acc_v, out_h.at[wid])

    return k(dst_flat)


def _tc_matmul(a, b, bm):
    m, k = a.shape
    _, n = b.shape

    def body(a_ref, b_ref, o_ref):
        o_ref[...] = jnp.dot(a_ref[...], b_ref[...], preferred_element_type=_F32)

    return pl.pallas_call(
        body,
        grid=(pl.cdiv(m, bm),),
        in_specs=[
            pl.BlockSpec((bm, k), lambda i: (i, 0)),
            pl.BlockSpec((k, n), lambda i: (0, 0)),
        ],
        out_specs=pl.BlockSpec((bm, n), lambda i: (i, 0)),
        out_shape=jax.ShapeDtypeStruct((m, n), _F32),
    )(a, b)


def _tc_dinv(degp):
    t, n_acc = degp.shape

    def body(d_ref, o_ref):
        deg = jnp.sum(d_ref[...], axis=0) + 1.0
        o_ref[...] = lax.rsqrt(deg).reshape(n_acc, 1)

    return pl.pallas_call(
        body,
        grid=(1,),
        in_specs=[pl.BlockSpec((t, n_acc), lambda i: (0, 0))],
        out_specs=pl.BlockSpec((n_acc, 1), lambda i: (0, 0)),
        out_shape=jax.ShapeDtypeStruct((n_acc, 1), _F32),
    )(degp)


def _tc_scale_h(dinv, h, bm):
    n, f = h.shape

    def body(d_ref, h_ref, o_ref):
        o_ref[...] = h_ref[...] * d_ref[...]

    return pl.pallas_call(
        body,
        grid=(n // bm,),
        in_specs=[
            pl.BlockSpec((bm, 1), lambda i: (i, 0)),
            pl.BlockSpec((bm, f), lambda i: (i, 0)),
        ],
        out_specs=pl.BlockSpec((bm, f), lambda i: (i, 0)),
        out_shape=jax.ShapeDtypeStruct((n, f), _F32),
    )(dinv, h)


def _tc_gcn_combine(dinv, a0, a1, h, b, bm):
    n, f = h.shape

    def body(d_ref, a0_ref, a1_ref, h_ref, b_ref, o_ref):
        dinv_c = d_ref[...]
        agg = a0_ref[...] + a1_ref[...]
        o_ref[...] = dinv_c * agg + (dinv_c * dinv_c) * h_ref[...] + b_ref[...]

    return pl.pallas_call(
        body,
        grid=(n // bm,),
        in_specs=[
            pl.BlockSpec((bm, 1), lambda i: (i, 0)),
            pl.BlockSpec((bm, f), lambda i: (i, 0)),
            pl.BlockSpec((bm, f), lambda i: (i, 0)),
            pl.BlockSpec((bm, f), lambda i: (i, 0)),
            pl.BlockSpec((1, f), lambda i: (0, 0)),
        ],
        out_specs=pl.BlockSpec((bm, f), lambda i: (i, 0)),
        out_shape=jax.ShapeDtypeStruct((n, f), _F32),
    )(dinv, a0, a1, h, b)


def _tc_gin(p, a0, a1, w1, b1, w2, b2, bm):
    n, f = p.shape

    def body(p_ref, a0_ref, a1_ref, w1_ref, b1_ref, w2_ref, b2_ref, o_ref):
        y = p_ref[...] + a0_ref[...] + a1_ref[...]
        t = jnp.maximum(jnp.dot(y, w1_ref[...], preferred_element_type=_F32)
                        + b1_ref[...], 0.0)
        z = jnp.maximum(jnp.dot(t, w2_ref[...], preferred_element_type=_F32)
                        + b2_ref[...], 0.0)
        o_ref[...] = z

    return pl.pallas_call(
        body,
        grid=(n // bm,),
        in_specs=[
            pl.BlockSpec((bm, f), lambda i: (i, 0)),
            pl.BlockSpec((bm, f), lambda i: (i, 0)),
            pl.BlockSpec((bm, f), lambda i: (i, 0)),
            pl.BlockSpec((f, f), lambda i: (0, 0)),
            pl.BlockSpec((1, f), lambda i: (0, 0)),
            pl.BlockSpec((f, f), lambda i: (0, 0)),
            pl.BlockSpec((1, f), lambda i: (0, 0)),
        ],
        out_specs=pl.BlockSpec((bm, f), lambda i: (i, 0)),
        out_shape=jax.ShapeDtypeStruct((n, f), _F32),
    )(p, a0, a1, w1, b1, w2, b2)


def _tc_head(p, a0, a1, w1, b1, w2, b2, w_att, lin_w, lin_b):
    n, f = p.shape

    def body(p_ref, a0_ref, a1_ref, w1_ref, b1_ref, w2_ref, b2_ref,
             wa_ref, lw_ref, lb_ref, o_ref):
        y = p_ref[...] + a0_ref[...] + a1_ref[...]
        t = jnp.maximum(jnp.dot(y, w1_ref[...], preferred_element_type=_F32)
                        + b1_ref[...], 0.0)
        hg = jnp.maximum(jnp.dot(t, w2_ref[...], preferred_element_type=_F32)
                         + b2_ref[...], 0.0)
        s = jnp.dot(hg, wa_ref[...], preferred_element_type=_F32)
        rep = lax.dot_general(s, hg, (((0,), (0,)), ((), ())),
                              preferred_element_type=_F32)
        o_ref[...] = jnp.dot(rep, lw_ref[...], preferred_element_type=_F32) \
            + lb_ref[...]

    return pl.pallas_call(
        body,
        grid=(1,),
        in_specs=[
            pl.BlockSpec((n, f), lambda i: (0, 0)),
            pl.BlockSpec((n, f), lambda i: (0, 0)),
            pl.BlockSpec((n, f), lambda i: (0, 0)),
            pl.BlockSpec((f, f), lambda i: (0, 0)),
            pl.BlockSpec((1, f), lambda i: (0, 0)),
            pl.BlockSpec((f, f), lambda i: (0, 0)),
            pl.BlockSpec((1, f), lambda i: (0, 0)),
            pl.BlockSpec((f, 1), lambda i: (0, 0)),
            pl.BlockSpec((f, 2), lambda i: (0, 0)),
            pl.BlockSpec((1, 2), lambda i: (0, 0)),
        ],
        out_specs=pl.BlockSpec((1, 2), lambda i: (0, 0)),
        out_shape=jax.ShapeDtypeStruct((1, 2), _F32),
    )(p, a0, a1, w1, b1, w2, b2, w_att, lin_w, lin_b)


def _ceil_batches(e):
    nb = -(-e // _BATCH)
    return -(-nb // (_NTILES * 8)) * (_NTILES * 8)


def kernel(x, edge_index, RT_mat0, RT_edge0, RT_mat1, RT_edge1, gcn_W, gcn_b,
           g0_W1, g0_b1, g0_W2, g0_b2, g1_W1, g1_b1, g1_W2, g1_b2,
           w_att, lin_W, lin_b):
    n, f = x.shape
    n1 = RT_mat0.shape[0]
    n2 = RT_mat1.shape[0]
    n_acc = -(-(n + 1) // 128) * 128 + 128
    n1_acc = -(-(n1 + 1) // 128) * 128 + 128
    n2_acc = -(-(n2 + 1) // 128) * 128 + 128

    src2d, dst2d = _pad_edges(edge_index[0], edge_index[1],
                              _ceil_batches(edge_index.shape[1]), n, n_acc - n)
    src2d_0, dst2d_0 = _pad_edges(RT_edge0[0], RT_edge0[1],
                                  _ceil_batches(RT_edge0.shape[1]), n1,
                                  n1_acc - n1)
    src2d_1, dst2d_1 = _pad_edges(RT_edge1[0], RT_edge1[1],
                                  _ceil_batches(RT_edge1.shape[1]), n2,
                                  n2_acc - n2)

    degp = _sc_degree(dst2d.reshape(-1), n_acc)
    dinv = _tc_dinv(degp)
    h = _tc_matmul(x, gcn_W, bm=2000)
    h2 = _tc_scale_h(dinv[:n], h, bm=2000)
    aggp = _sc_segment_sum(h2, src2d, dst2d, n_acc, f, nbuf=2)
    gcn = _tc_gcn_combine(dinv[:n], aggp[0, :n], aggp[1, :n],
                          h, gcn_b.reshape(1, f), bm=2000)

    p0 = _tc_matmul(RT_mat0, gcn, bm=512)
    agg0 = _sc_segment_sum(p0, src2d_0, dst2d_0, n1_acc, f, nbuf=2)
    h1 = _tc_gin(p0, agg0[0, :n1], agg0[1, :n1], g0_W1, g0_b1.reshape(1, f),
                 g0_W2, g0_b2.reshape(1, f), bm=n1)

    p1 = _tc_matmul(RT_mat1, h1, bm=n2)
    agg1 = _sc_segment_sum(p1, src2d_1, dst2d_1, n2_acc, f, nbuf=2)
    score = _tc_head(p1, agg1[0, :n2], agg1[1, :n2], g1_W1, g1_b1.reshape(1, f),
                     g1_W2, g1_b2.reshape(1, f), w_att, lin_W,
                     lin_b.reshape(1, 2))
    return score


# speedup vs baseline: 5.4827x; 5.4827x over previous
import functools

import jax
import jax.numpy as jnp
from jax import lax
from jax.experimental import pallas as pl
from jax.experimental.pallas import tpu as pltpu
from jax.experimental.pallas import tpu_sc as plsc

_F32 = jnp.float32
_BATCH = 128
_NTILES = 32


def _pad_edges(src, dst, n_batches, dummy_row, n_dummy):
    e = src.shape[0]
    pad = n_batches * _BATCH - e
    src_fill = jnp.arange(pad, dtype=jnp.int32) % dummy_row
    src_p = jnp.concatenate([src, src_fill])
    dst_fill = dummy_row + (jnp.arange(pad, dtype=jnp.int32) % n_dummy)
    dst_p = jnp.concatenate([dst, dst_fill])
    return src_p.reshape(n_batches, _BATCH), dst_p.reshape(n_batches, _BATCH)


def _sc_segment_sum(table, src2d, dst2d, n_rows_acc, feat, nbuf):
    n_batches = src2d.shape[0]
    nb_per_tile = n_batches // _NTILES
    nring = 2 * nbuf
    assert nb_per_tile % nring == 0
    chunk = n_rows_acc // 16
    zeros = jnp.zeros((chunk, feat), _F32)
    mesh = plsc.VectorSubcoreMesh(core_axis_name="c", subcore_axis_name="s")

    @functools.partial(
        pl.kernel,
        mesh=mesh,
        out_type=jax.ShapeDtypeStruct((2, n_rows_acc, feat), _F32),
        scratch_types=[
            pltpu.VMEM((nring, _BATCH), jnp.int32),
            pltpu.VMEM((nring, _BATCH), jnp.int32),
            pltpu.VMEM((nbuf, _BATCH, feat), _F32),
            pltpu.VMEM_SHARED((n_rows_acc, feat), _F32),
            [pltpu.SemaphoreType.DMA] * nring,
            [pltpu.SemaphoreType.DMA] * nbuf,
            [pltpu.SemaphoreType.DMA] * nbuf,
        ],
    )
    def k(table_h, src_h, dst_h, zeros_h, out_h, src_v, dst_v, rows_v, acc,
          isems, gsems, ssems):
        c = lax.axis_index("c")
        s = lax.axis_index("s")
        wid = c * 16 + s
        base = wid * nb_per_tile

        def idx_start(j, slot):
            off = (base + j) * _BATCH
            pltpu.async_copy(
                src_h.at[pl.ds(off, _BATCH)], src_v.at[slot], isems[slot])
            pltpu.async_copy(
                dst_h.at[pl.ds(off, _BATCH)], dst_v.at[slot], isems[slot])

        def idx_wait(j, slot):
            off = (base + j) * _BATCH
            pltpu.make_async_copy(
                src_h.at[pl.ds(off, _BATCH)], src_v.at[slot],
                isems[slot]).wait()
            pltpu.make_async_copy(
                dst_h.at[pl.ds(off, _BATCH)], dst_v.at[slot],
                isems[slot]).wait()

        pltpu.sync_copy(zeros_h, acc.at[pl.ds(s * chunk, chunk)])
        for m in range(nring):
            idx_start(m, m)
        for b in range(nbuf):
            idx_wait(b, b)
            pltpu.async_copy(table_h.at[src_v.at[b]], rows_v.at[b], gsems[b])
        plsc.subcore_barrier()

        def body(g, carry):
            j0 = g * nring
            for p in range(2):
                for b in range(nbuf):
                    j = j0 + p * nbuf + b
                    slot = p * nbuf + b
                    pltpu.make_async_copy(
                        table_h.at[src_v.at[slot]], rows_v.at[b],
                        gsems[b]).wait()
                    pltpu.async_copy(
                        rows_v.at[b], acc.at[dst_v.at[slot]], ssems[b],
                        add=True)
                for b in range(nbuf):
                    j = j0 + p * nbuf + b
                    slot = p * nbuf + b
                    nslot = (slot + nbuf) % nring
                    pltpu.make_async_copy(
                        rows_v.at[b], acc.at[dst_v.at[slot]], ssems[b]).wait()
                    nj = j + nbuf

                    @pl.when(nj < nb_per_tile)
                    def _():
                        idx_wait(nj, nslot)
                        pltpu.async_copy(
                            table_h.at[src_v.at[nslot]], rows_v.at[b],
                            gsems[b])

                    nnj = j + nring

                    @pl.when(nnj < nb_per_tile)
                    def _():
                        idx_start(nnj, slot)
            return carry

        lax.fori_loop(0, nb_per_tile // nring, body, 0)
        plsc.subcore_barrier()
        pltpu.sync_copy(acc.at[pl.ds(s * chunk, chunk)],
                        out_h.at[c].at[pl.ds(s * chunk, chunk)])

    return k(table, src2d.reshape(-1), dst2d.reshape(-1), zeros)


def _sc_degree(dst_flat, n_rows_acc):
    n_batches = dst_flat.shape[0] // _BATCH
    nb_per_tile = n_batches // _NTILES
    e_per_tile = nb_per_tile * _BATCH
    mesh = plsc.VectorSubcoreMesh(core_axis_name="c", subcore_axis_name="s")

    @functools.partial(
        pl.kernel,
        mesh=mesh,
        out_type=jax.ShapeDtypeStruct((_NTILES, n_rows_acc), _F32),
        scratch_types=[
            pltpu.VMEM((e_per_tile,), jnp.int32),
            pltpu.VMEM((n_rows_acc,), _F32),
        ],
        compiler_params=pltpu.CompilerParams(needs_layout_passes=False),
    )
    def k(dst_h, out_h, dst_v, acc_v):
        c = lax.axis_index("c")
        s = lax.axis_index("s")
        wid = c * 16 + s
        pltpu.sync_copy(dst_h.at[pl.ds(wid * e_per_tile, e_per_tile)], dst_v)
        zero16 = jnp.zeros((16,), _F32)

        def zbody(i, carry):
            acc_v[pl.ds(i * 16, 16)] = zero16
            return carry

        lax.fori_loop(0, n_rows_acc // 16, zbody, 0)
        one16 = jnp.ones((16,), _F32)

        def body(j, carry):
            idx = dst_v[pl.ds(j * 16, 16)]
            plsc.addupdate_scatter(acc_v, [idx], one16)
            return carry

        lax.fori_loop(0, e_per_tile // 16, body, 0)
        pltpu.sync_copy(acc_v, out_h.at[wid])

    return k(dst_flat)


def _tc_matmul(a, b, bm):
    m, k = a.shape
    _, n = b.shape

    def body(a_ref, b_ref, o_ref):
        o_ref[...] = jnp.dot(a_ref[...], b_ref[...], preferred_element_type=_F32)

    return pl.pallas_call(
        body,
        grid=(pl.cdiv(m, bm),),
        in_specs=[
            pl.BlockSpec((bm, k), lambda i: (i, 0)),
            pl.BlockSpec((k, n), lambda i: (0, 0)),
        ],
        out_specs=pl.BlockSpec((bm, n), lambda i: (i, 0)),
        out_shape=jax.ShapeDtypeStruct((m, n), _F32),
    )(a, b)


def _tc_dinv(degp):
    t, n_acc = degp.shape

    def body(d_ref, o_ref):
        deg = jnp.sum(d_ref[...], axis=0) + 1.0
        o_ref[...] = lax.rsqrt(deg).reshape(n_acc, 1)

    return pl.pallas_call(
        body,
        grid=(1,),
        in_specs=[pl.BlockSpec((t, n_acc), lambda i: (0, 0))],
        out_specs=pl.BlockSpec((n_acc, 1), lambda i: (0, 0)),
        out_shape=jax.ShapeDtypeStruct((n_acc, 1), _F32),
    )(degp)


def _tc_scale_h(dinv, h, bm):
    n, f = h.shape

    def body(d_ref, h_ref, o_ref):
        o_ref[...] = h_ref[...] * d_ref[...]

    return pl.pallas_call(
        body,
        grid=(n // bm,),
        in_specs=[
            pl.BlockSpec((bm, 1), lambda i: (i, 0)),
            pl.BlockSpec((bm, f), lambda i: (i, 0)),
        ],
        out_specs=pl.BlockSpec((bm, f), lambda i: (i, 0)),
        out_shape=jax.ShapeDtypeStruct((n, f), _F32),
    )(dinv, h)


def _tc_gcn_combine(dinv, a0, a1, h, b, bm):
    n, f = h.shape

    def body(d_ref, a0_ref, a1_ref, h_ref, b_ref, o_ref):
        dinv_c = d_ref[...]
        agg = a0_ref[...] + a1_ref[...]
        o_ref[...] = dinv_c * agg + (dinv_c * dinv_c) * h_ref[...] + b_ref[...]

    return pl.pallas_call(
        body,
        grid=(n // bm,),
        in_specs=[
            pl.BlockSpec((bm, 1), lambda i: (i, 0)),
            pl.BlockSpec((bm, f), lambda i: (i, 0)),
            pl.BlockSpec((bm, f), lambda i: (i, 0)),
            pl.BlockSpec((bm, f), lambda i: (i, 0)),
            pl.BlockSpec((1, f), lambda i: (0, 0)),
        ],
        out_specs=pl.BlockSpec((bm, f), lambda i: (i, 0)),
        out_shape=jax.ShapeDtypeStruct((n, f), _F32),
    )(dinv, a0, a1, h, b)


def _tc_gin(p, a0, a1, w1, b1, w2, b2, bm):
    n, f = p.shape

    def body(p_ref, a0_ref, a1_ref, w1_ref, b1_ref, w2_ref, b2_ref, o_ref):
        y = p_ref[...] + a0_ref[...] + a1_ref[...]
        t = jnp.maximum(jnp.dot(y, w1_ref[...], preferred_element_type=_F32)
                        + b1_ref[...], 0.0)
        z = jnp.maximum(jnp.dot(t, w2_ref[...], preferred_element_type=_F32)
                        + b2_ref[...], 0.0)
        o_ref[...] = z

    return pl.pallas_call(
        body,
        grid=(n // bm,),
        in_specs=[
            pl.BlockSpec((bm, f), lambda i: (i, 0)),
            pl.BlockSpec((bm, f), lambda i: (i, 0)),
            pl.BlockSpec((bm, f), lambda i: (i, 0)),
            pl.BlockSpec((f, f), lambda i: (0, 0)),
            pl.BlockSpec((1, f), lambda i: (0, 0)),
            pl.BlockSpec((f, f), lambda i: (0, 0)),
            pl.BlockSpec((1, f), lambda i: (0, 0)),
        ],
        out_specs=pl.BlockSpec((bm, f), lambda i: (i, 0)),
        out_shape=jax.ShapeDtypeStruct((n, f), _F32),
    )(p, a0, a1, w1, b1, w2, b2)


def _tc_head(p, a0, a1, w1, b1, w2, b2, w_att, lin_w, lin_b):
    n, f = p.shape

    def body(p_ref, a0_ref, a1_ref, w1_ref, b1_ref, w2_ref, b2_ref,
             wa_ref, lw_ref, lb_ref, o_ref):
        y = p_ref[...] + a0_ref[...] + a1_ref[...]
        t = jnp.maximum(jnp.dot(y, w1_ref[...], preferred_element_type=_F32)
                        + b1_ref[...], 0.0)
        hg = jnp.maximum(jnp.dot(t, w2_ref[...], preferred_element_type=_F32)
                         + b2_ref[...], 0.0)
        s = jnp.dot(hg, wa_ref[...], preferred_element_type=_F32)
        rep = lax.dot_general(s, hg, (((0,), (0,)), ((), ())),
                              preferred_element_type=_F32)
        o_ref[...] = jnp.dot(rep, lw_ref[...], preferred_element_type=_F32) \
            + lb_ref[...]

    return pl.pallas_call(
        body,
        grid=(1,),
        in_specs=[
            pl.BlockSpec((n, f), lambda i: (0, 0)),
            pl.BlockSpec((n, f), lambda i: (0, 0)),
            pl.BlockSpec((n, f), lambda i: (0, 0)),
            pl.BlockSpec((f, f), lambda i: (0, 0)),
            pl.BlockSpec((1, f), lambda i: (0, 0)),
            pl.BlockSpec((f, f), lambda i: (0, 0)),
            pl.BlockSpec((1, f), lambda i: (0, 0)),
            pl.BlockSpec((f, 1), lambda i: (0, 0)),
            pl.BlockSpec((f, 2), lambda i: (0, 0)),
            pl.BlockSpec((1, 2), lambda i: (0, 0)),
        ],
        out_specs=pl.BlockSpec((1, 2), lambda i: (0, 0)),
        out_shape=jax.ShapeDtypeStruct((1, 2), _F32),
    )(p, a0, a1, w1, b1, w2, b2, w_att, lin_w, lin_b)


def _ceil_batches(e):
    nb = -(-e // _BATCH)
    return -(-nb // (_NTILES * 8)) * (_NTILES * 8)


def kernel(x, edge_index, RT_mat0, RT_edge0, RT_mat1, RT_edge1, gcn_W, gcn_b,
           g0_W1, g0_b1, g0_W2, g0_b2, g1_W1, g1_b1, g1_W2, g1_b2,
           w_att, lin_W, lin_b):
    n, f = x.shape
    n1 = RT_mat0.shape[0]
    n2 = RT_mat1.shape[0]
    n_acc = -(-(n + 1) // 128) * 128 + 128
    n1_acc = -(-(n1 + 1) // 128) * 128 + 128
    n2_acc = -(-(n2 + 1) // 128) * 128 + 128

    src2d, dst2d = _pad_edges(edge_index[0], edge_index[1],
                              _ceil_batches(edge_index.shape[1]), n, n_acc - n)
    src2d_0, dst2d_0 = _pad_edges(RT_edge0[0], RT_edge0[1],
                                  _ceil_batches(RT_edge0.shape[1]), n1,
                                  n1_acc - n1)
    src2d_1, dst2d_1 = _pad_edges(RT_edge1[0], RT_edge1[1],
                                  _ceil_batches(RT_edge1.shape[1]), n2,
                                  n2_acc - n2)

    degp = _sc_degree(dst2d.reshape(-1), n_acc)
    dinv = _tc_dinv(degp)
    h = _tc_matmul(x, gcn_W, bm=2000)
    h2 = _tc_scale_h(dinv[:n], h, bm=2000)
    aggp = _sc_segment_sum(h2, src2d, dst2d, n_acc, f, nbuf=2)
    gcn = _tc_gcn_combine(dinv[:n], aggp[0, :n], aggp[1, :n],
                          h, gcn_b.reshape(1, f), bm=2000)

    p0 = _tc_matmul(RT_mat0, gcn, bm=512)
    agg0 = _sc_segment_sum(p0, src2d_0, dst2d_0, n1_acc, f, nbuf=2)
    h1 = _tc_gin(p0, agg0[0, :n1], agg0[1, :n1], g0_W1, g0_b1.reshape(1, f),
                 g0_W2, g0_b2.reshape(1, f), bm=n1)

    p1 = _tc_matmul(RT_mat1, h1, bm=n2)
    agg1 = _sc_segment_sum(p1, src2d_1, dst2d_1, n2_acc, f, nbuf=2)
    score = _tc_head(p1, agg1[0, :n2], agg1[1, :n2], g1_W1, g1_b1.reshape(1, f),
                     g1_W2, g1_b2.reshape(1, f), w_att, lin_W,
                     lin_b.reshape(1, 2))
    return score


# speedup vs baseline: 5.5968x; 1.0208x over previous
import functools

import jax
import jax.numpy as jnp
from jax import lax
from jax.experimental import pallas as pl
from jax.experimental.pallas import tpu as pltpu
from jax.experimental.pallas import tpu_sc as plsc

_F32 = jnp.float32
_BATCH = 128
_NTILES = 32


def _pad_edges(src, dst, n_batches, dummy_row, n_dummy):
    e = src.shape[0]
    pad = n_batches * _BATCH - e
    src_fill = jnp.arange(pad, dtype=jnp.int32) % dummy_row
    src_p = jnp.concatenate([src, src_fill])
    dst_fill = dummy_row + (jnp.arange(pad, dtype=jnp.int32) % n_dummy)
    dst_p = jnp.concatenate([dst, dst_fill])
    return src_p.reshape(n_batches, _BATCH), dst_p.reshape(n_batches, _BATCH)


def _sc_segment_sum(table, src2d, dst2d, n_rows_acc, feat, nbuf):
    n_batches = src2d.shape[0]
    nb_per_tile = n_batches // _NTILES
    nring = 2 * nbuf
    assert nb_per_tile % nring == 0
    chunk = n_rows_acc // 16
    zeros = jnp.zeros((chunk, feat), _F32)
    mesh = plsc.VectorSubcoreMesh(core_axis_name="c", subcore_axis_name="s")

    @functools.partial(
        pl.kernel,
        mesh=mesh,
        out_type=jax.ShapeDtypeStruct((2, n_rows_acc, feat), _F32),
        scratch_types=[
            pltpu.VMEM((nring, _BATCH), jnp.int32),
            pltpu.VMEM((nring, _BATCH), jnp.int32),
            pltpu.VMEM((nbuf, _BATCH, feat), _F32),
            pltpu.VMEM_SHARED((n_rows_acc, feat), _F32),
            [pltpu.SemaphoreType.DMA] * nring,
            [pltpu.SemaphoreType.DMA] * nbuf,
            [pltpu.SemaphoreType.DMA] * nbuf,
        ],
    )
    def k(table_h, src_h, dst_h, zeros_h, out_h, src_v, dst_v, rows_v, acc,
          isems, gsems, ssems):
        c = lax.axis_index("c")
        s = lax.axis_index("s")
        wid = c * 16 + s
        base = wid * nb_per_tile

        def idx_start(j, slot):
            off = (base + j) * _BATCH
            pltpu.async_copy(
                src_h.at[pl.ds(off, _BATCH)], src_v.at[slot], isems[slot])
            pltpu.async_copy(
                dst_h.at[pl.ds(off, _BATCH)], dst_v.at[slot], isems[slot])

        def idx_wait(j, slot):
            off = (base + j) * _BATCH
            pltpu.make_async_copy(
                src_h.at[pl.ds(off, _BATCH)], src_v.at[slot],
                isems[slot]).wait()
            pltpu.make_async_copy(
                dst_h.at[pl.ds(off, _BATCH)], dst_v.at[slot],
                isems[slot]).wait()

        pltpu.sync_copy(zeros_h, acc.at[pl.ds(s * chunk, chunk)])
        for m in range(nring):
            idx_start(m, m)
        for b in range(nbuf):
            idx_wait(b, b)
            pltpu.async_copy(table_h.at[src_v.at[b]], rows_v.at[b], gsems[b])
        plsc.subcore_barrier()

        def body(g, carry):
            j0 = g * nring
            for p in range(2):
                for b in range(nbuf):
                    j = j0 + p * nbuf + b
                    slot = p * nbuf + b
                    pltpu.make_async_copy(
                        table_h.at[src_v.at[slot]], rows_v.at[b],
                        gsems[b]).wait()
                    pltpu.async_copy(
                        rows_v.at[b], acc.at[dst_v.at[slot]], ssems[b],
                        add=True)
                for b in range(nbuf):
                    j = j0 + p * nbuf + b
                    slot = p * nbuf + b
                    nslot = (slot + nbuf) % nring
                    pltpu.make_async_copy(
                        rows_v.at[b], acc.at[dst_v.at[slot]], ssems[b]).wait()
                    nj = j + nbuf

                    @pl.when(nj < nb_per_tile)
                    def _():
                        idx_wait(nj, nslot)
                        pltpu.async_copy(
                            table_h.at[src_v.at[nslot]], rows_v.at[b],
                            gsems[b])

                    nnj = j + nring

                    @pl.when(nnj < nb_per_tile)
                    def _():
                        idx_start(nnj, slot)
            return carry

        lax.fori_loop(0, nb_per_tile // nring, body, 0)
        plsc.subcore_barrier()
        pltpu.sync_copy(acc.at[pl.ds(s * chunk, chunk)],
                        out_h.at[c].at[pl.ds(s * chunk, chunk)])

    return k(table, src2d.reshape(-1), dst2d.reshape(-1), zeros)


def _sc_degree(dst_flat, n_rows_acc):
    n_batches = dst_flat.shape[0] // _BATCH
    nb_per_tile = n_batches // _NTILES
    e_per_tile = nb_per_tile * _BATCH
    mesh = plsc.VectorSubcoreMesh(core_axis_name="c", subcore_axis_name="s")

    @functools.partial(
        pl.kernel,
        mesh=mesh,
        out_type=jax.ShapeDtypeStruct((_NTILES, n_rows_acc), _F32),
        scratch_types=[
            pltpu.VMEM((e_per_tile,), jnp.int32),
            pltpu.VMEM((n_rows_acc,), _F32),
        ],
        compiler_params=pltpu.CompilerParams(needs_layout_passes=False),
    )
    def k(dst_h, out_h, dst_v, acc_v):
        c = lax.axis_index("c")
        s = lax.axis_index("s")
        wid = c * 16 + s
        pltpu.sync_copy(dst_h.at[pl.ds(wid * e_per_tile, e_per_tile)], dst_v)
        zero16 = jnp.zeros((16,), _F32)

        def zbody(i, carry):
            acc_v[pl.ds(i * 16, 16)] = zero16
            return carry

        lax.fori_loop(0, n_rows_acc // 16, zbody, 0)
        one16 = jnp.ones((16,), _F32)

        def body(j, carry):
            idx = dst_v[pl.ds(j * 16, 16)]
            plsc.addupdate_scatter(acc_v, [idx], one16)
            return carry

        lax.fori_loop(0, e_per_tile // 16, body, 0)
        pltpu.sync_copy(acc_v, out_h.at[wid])

    return k(dst_flat)


def _tc_matmul(a, b, bm):
    m, k = a.shape
    _, n = b.shape

    def body(a_ref, b_ref, o_ref):
        o_ref[...] = jnp.dot(a_ref[...], b_ref[...], preferred_element_type=_F32)

    return pl.pallas_call(
        body,
        grid=(pl.cdiv(m, bm),),
        in_specs=[
            pl.BlockSpec((bm, k), lambda i: (i, 0)),
            pl.BlockSpec((k, n), lambda i: (0, 0)),
        ],
        out_specs=pl.BlockSpec((bm, n), lambda i: (i, 0)),
        out_shape=jax.ShapeDtypeStruct((m, n), _F32),
    )(a, b)


def _tc_dinv(degp):
    t, n_acc = degp.shape

    def body(d_ref, o_ref):
        deg = jnp.sum(d_ref[...], axis=0) + 1.0
        o_ref[...] = lax.rsqrt(deg).reshape(n_acc, 1)

    return pl.pallas_call(
        body,
        grid=(1,),
        in_specs=[pl.BlockSpec((t, n_acc), lambda i: (0, 0))],
        out_specs=pl.BlockSpec((n_acc, 1), lambda i: (0, 0)),
        out_shape=jax.ShapeDtypeStruct((n_acc, 1), _F32),
    )(degp)


def _tc_scale_h(dinv, h, bm):
    n, f = h.shape

    def body(d_ref, h_ref, o_ref):
        o_ref[...] = h_ref[...] * d_ref[...]

    return pl.pallas_call(
        body,
        grid=(n // bm,),
        in_specs=[
            pl.BlockSpec((bm, 1), lambda i: (i, 0)),
            pl.BlockSpec((bm, f), lambda i: (i, 0)),
        ],
        out_specs=pl.BlockSpec((bm, f), lambda i: (i, 0)),
        out_shape=jax.ShapeDtypeStruct((n, f), _F32),
    )(dinv, h)


def _tc_gcn_combine(dinv, a0, a1, h, b, bm):
    n, f = h.shape

    def body(d_ref, a0_ref, a1_ref, h_ref, b_ref, o_ref):
        dinv_c = d_ref[...]
        agg = a0_ref[...] + a1_ref[...]
        o_ref[...] = dinv_c * agg + (dinv_c * dinv_c) * h_ref[...] + b_ref[...]

    return pl.pallas_call(
        body,
        grid=(n // bm,),
        in_specs=[
            pl.BlockSpec((bm, 1), lambda i: (i, 0)),
            pl.BlockSpec((bm, f), lambda i: (i, 0)),
            pl.BlockSpec((bm, f), lambda i: (i, 0)),
            pl.BlockSpec((bm, f), lambda i: (i, 0)),
            pl.BlockSpec((1, f), lambda i: (0, 0)),
        ],
        out_specs=pl.BlockSpec((bm, f), lambda i: (i, 0)),
        out_shape=jax.ShapeDtypeStruct((n, f), _F32),
    )(dinv, a0, a1, h, b)


def _tc_gin(p, a0, a1, w1, b1, w2, b2, bm):
    n, f = p.shape

    def body(p_ref, a0_ref, a1_ref, w1_ref, b1_ref, w2_ref, b2_ref, o_ref):
        y = p_ref[...] + a0_ref[...] + a1_ref[...]
        t = jnp.maximum(jnp.dot(y, w1_ref[...], preferred_element_type=_F32)
                        + b1_ref[...], 0.0)
        z = jnp.maximum(jnp.dot(t, w2_ref[...], preferred_element_type=_F32)
                        + b2_ref[...], 0.0)
        o_ref[...] = z

    return pl.pallas_call(
        body,
        grid=(n // bm,),
        in_specs=[
            pl.BlockSpec((bm, f), lambda i: (i, 0)),
            pl.BlockSpec((bm, f), lambda i: (i, 0)),
            pl.BlockSpec((bm, f), lambda i: (i, 0)),
            pl.BlockSpec((f, f), lambda i: (0, 0)),
            pl.BlockSpec((1, f), lambda i: (0, 0)),
            pl.BlockSpec((f, f), lambda i: (0, 0)),
            pl.BlockSpec((1, f), lambda i: (0, 0)),
        ],
        out_specs=pl.BlockSpec((bm, f), lambda i: (i, 0)),
        out_shape=jax.ShapeDtypeStruct((n, f), _F32),
    )(p, a0, a1, w1, b1, w2, b2)


def _tc_head(p, a0, a1, w1, b1, w2, b2, w_att, lin_w, lin_b):
    n, f = p.shape

    def body(p_ref, a0_ref, a1_ref, w1_ref, b1_ref, w2_ref, b2_ref,
             wa_ref, lw_ref, lb_ref, o_ref):
        y = p_ref[...] + a0_ref[...] + a1_ref[...]
        t = jnp.maximum(jnp.dot(y, w1_ref[...], preferred_element_type=_F32)
                        + b1_ref[...], 0.0)
        hg = jnp.maximum(jnp.dot(t, w2_ref[...], preferred_element_type=_F32)
                         + b2_ref[...], 0.0)
        s = jnp.dot(hg, wa_ref[...], preferred_element_type=_F32)
        rep = lax.dot_general(s, hg, (((0,), (0,)), ((), ())),
                              preferred_element_type=_F32)
        o_ref[...] = jnp.dot(rep, lw_ref[...], preferred_element_type=_F32) \
            + lb_ref[...]

    return pl.pallas_call(
        body,
        grid=(1,),
        in_specs=[
            pl.BlockSpec((n, f), lambda i: (0, 0)),
            pl.BlockSpec((n, f), lambda i: (0, 0)),
            pl.BlockSpec((n, f), lambda i: (0, 0)),
            pl.BlockSpec((f, f), lambda i: (0, 0)),
            pl.BlockSpec((1, f), lambda i: (0, 0)),
            pl.BlockSpec((f, f), lambda i: (0, 0)),
            pl.BlockSpec((1, f), lambda i: (0, 0)),
            pl.BlockSpec((f, 1), lambda i: (0, 0)),
            pl.BlockSpec((f, 2), lambda i: (0, 0)),
            pl.BlockSpec((1, 2), lambda i: (0, 0)),
        ],
        out_specs=pl.BlockSpec((1, 2), lambda i: (0, 0)),
        out_shape=jax.ShapeDtypeStruct((1, 2), _F32),
    )(p, a0, a1, w1, b1, w2, b2, w_att, lin_w, lin_b)


def _ceil_batches(e):
    nb = -(-e // _BATCH)
    return -(-nb // (_NTILES * 8)) * (_NTILES * 8)


def kernel(x, edge_index, RT_mat0, RT_edge0, RT_mat1, RT_edge1, gcn_W, gcn_b,
           g0_W1, g0_b1, g0_W2, g0_b2, g1_W1, g1_b1, g1_W2, g1_b2,
           w_att, lin_W, lin_b):
    n, f = x.shape
    n1 = RT_mat0.shape[0]
    n2 = RT_mat1.shape[0]
    n_acc = -(-(n + 1) // 128) * 128 + 128
    n1_acc = -(-(n1 + 1) // 128) * 128 + 128
    n2_acc = -(-(n2 + 1) // 128) * 128 + 128

    src2d, dst2d = _pad_edges(edge_index[0], edge_index[1],
                              _ceil_batches(edge_index.shape[1]), n, n_acc - n)
    src2d_0, dst2d_0 = _pad_edges(RT_edge0[0], RT_edge0[1],
                                  _ceil_batches(RT_edge0.shape[1]), n1,
                                  n1_acc - n1)
    src2d_1, dst2d_1 = _pad_edges(RT_edge1[0], RT_edge1[1],
                                  _ceil_batches(RT_edge1.shape[1]), n2,
                                  n2_acc - n2)

    degp = _sc_degree(dst2d.reshape(-1), n_acc)
    dinv = _tc_dinv(degp)
    h = _tc_matmul(x, gcn_W, bm=2000)
    h2 = _tc_scale_h(dinv[:n], h, bm=2000)
    aggp = _sc_segment_sum(h2, src2d, dst2d, n_acc, f, nbuf=2)
    gcn = _tc_gcn_combine(dinv[:n], aggp[0, :n], aggp[1, :n],
                          h, gcn_b.reshape(1, f), bm=2000)

    p0 = _tc_matmul(RT_mat0, gcn, bm=512)
    agg0 = _sc_segment_sum(p0, src2d_0, dst2d_0, n1_acc, f, nbuf=4)
    h1 = _tc_gin(p0, agg0[0, :n1], agg0[1, :n1], g0_W1, g0_b1.reshape(1, f),
                 g0_W2, g0_b2.reshape(1, f), bm=n1)

    p1 = _tc_matmul(RT_mat1, h1, bm=n2)
    agg1 = _sc_segment_sum(p1, src2d_1, dst2d_1, n2_acc, f, nbuf=4)
    score = _tc_head(p1, agg1[0, :n2], agg1[1, :n2], g1_W1, g1_b1.reshape(1, f),
                     g1_W2, g1_b2.reshape(1, f), w_att, lin_W,
                     lin_b.reshape(1, 2))
    return score
